# Initial kernel scaffold; baseline (speedup 1.0000x reference)
import jax
import jax.numpy as jnp
from jax.experimental import pallas as pl


def kernel(x, edge_index, a_src, a_dst, W_p, W_q, W_up0, W_up1, W_up2, W_down0, W_down1, W_down2, W_out, b_out):
    raise NotImplementedError("write your pallas kernel here")


# speedup vs baseline: 6.5172x; 6.5172x over previous
import functools

import jax
import jax.numpy as jnp
from jax import lax
from jax.experimental import pallas as pl
from jax.experimental.pallas import tpu as pltpu
from jax.experimental.pallas import tpu_sc as plsc

F32 = jnp.float32
I32 = jnp.int32
NC, NS, L = 2, 16, 16
NW = NC * NS
_HI = lax.Precision.HIGHEST


def _dot(a, b):
    return jnp.dot(a, b, precision=_HI, preferred_element_type=F32)


def _mesh():
    return plsc.VectorSubcoreMesh(
        core_axis_name="c", subcore_axis_name="s", num_cores=NC, num_subcores=NS
    )


_NB = 2000


def _tc_pre(x, asd, wp, wq, wu):
    n, d = x.shape
    h = wp.shape[1]

    def st_body(x_ref, asd_ref, st_ref):
        st_ref[...] = lax.dot_general(
            asd_ref[...], x_ref[...], (((1,), (1,)), ((), ())),
            precision=_HI, preferred_element_type=F32)

    st = pl.pallas_call(
        st_body,
        out_shape=jax.ShapeDtypeStruct((2, n), F32),
    )(x, asd)

    def body(x_ref, asd_ref, wp_ref, wq_ref, wu_ref, p_ref, q_ref, u_ref):
        xb = x_ref[...]
        p_ref[...] = _dot(xb, wp_ref[...])
        qb = _dot(xb, wq_ref[...])
        q_ref[...] = qb
        u_ref[...] = _dot(qb, wu_ref[...])

    p, q, u = pl.pallas_call(
        body,
        grid=(n // _NB,),
        in_specs=[
            pl.BlockSpec((_NB, d), lambda i: (i, 0)),
            pl.BlockSpec((2, d), lambda i: (0, 0)),
            pl.BlockSpec((d, h), lambda i: (0, 0)),
            pl.BlockSpec((d, h), lambda i: (0, 0)),
            pl.BlockSpec((h, h), lambda i: (0, 0)),
        ],
        out_specs=[
            pl.BlockSpec((_NB, h), lambda i: (i, 0)),
            pl.BlockSpec((_NB, h), lambda i: (i, 0)),
            pl.BlockSpec((_NB, h), lambda i: (i, 0)),
        ],
        out_shape=[
            jax.ShapeDtypeStruct((n, h), F32),
            jax.ShapeDtypeStruct((n, h), F32),
            jax.ShapeDtypeStruct((n, h), F32),
        ],
    )(x, asd, wp, wq, wu)
    return st, p, q, u


def _tc_mid_a(p, y, wd):
    n, h = p.shape

    def body(p_ref, y_ref, wd_ref, pn_ref, v_ref):
        pn = p_ref[...] + y_ref[0] + y_ref[1]
        pn_ref[...] = pn
        v_ref[...] = _dot(pn, wd_ref[...])

    return pl.pallas_call(
        body,
        grid=(n // _NB,),
        in_specs=[
            pl.BlockSpec((_NB, h), lambda i: (i, 0)),
            pl.BlockSpec((2, _NB, h), lambda i: (0, i, 0)),
            pl.BlockSpec((h, h), lambda i: (0, 0)),
        ],
        out_specs=[
            pl.BlockSpec((_NB, h), lambda i: (i, 0)),
            pl.BlockSpec((_NB, h), lambda i: (i, 0)),
        ],
        out_shape=[
            jax.ShapeDtypeStruct((n, h), F32),
            jax.ShapeDtypeStruct((n, h), F32),
        ],
    )(p, y, wd)


def _tc_mid_b(q, y, p, wu):
    n, h = q.shape

    def body(q_ref, y_ref, p_ref, wu_ref, pn_ref, qn_ref, u_ref):
        qn = jnp.tanh(q_ref[...] - (y_ref[0] + y_ref[1]))
        pn_ref[...] = jnp.tanh(p_ref[...])
        qn_ref[...] = qn
        u_ref[...] = _dot(qn, wu_ref[...])

    return pl.pallas_call(
        body,
        grid=(n // _NB,),
        in_specs=[
            pl.BlockSpec((_NB, h), lambda i: (i, 0)),
            pl.BlockSpec((2, _NB, h), lambda i: (0, i, 0)),
            pl.BlockSpec((_NB, h), lambda i: (i, 0)),
            pl.BlockSpec((h, h), lambda i: (0, 0)),
        ],
        out_specs=[
            pl.BlockSpec((_NB, h), lambda i: (i, 0)),
            pl.BlockSpec((_NB, h), lambda i: (i, 0)),
            pl.BlockSpec((_NB, h), lambda i: (i, 0)),
        ],
        out_shape=[
            jax.ShapeDtypeStruct((n, h), F32),
            jax.ShapeDtypeStruct((n, h), F32),
            jax.ShapeDtypeStruct((n, h), F32),
        ],
    )(q, y, p, wu)


def _tc_fin(q, y, wo, bo):
    n, h = q.shape
    c = wo.shape[1]

    def body(q_ref, y_ref, wo_ref, bo_ref, o_ref):
        qn = q_ref[...] - (y_ref[0] + y_ref[1])
        o_ref[...] = _dot(qn, wo_ref[...]) + bo_ref[...]

    return pl.pallas_call(
        body,
        grid=(n // _NB,),
        in_specs=[
            pl.BlockSpec((_NB, h), lambda i: (i, 0)),
            pl.BlockSpec((2, _NB, h), lambda i: (0, i, 0)),
            pl.BlockSpec((h, c), lambda i: (0, 0)),
            pl.BlockSpec((1, c), lambda i: (0, 0)),
        ],
        out_specs=pl.BlockSpec((_NB, c), lambda i: (i, 0)),
        out_shape=jax.ShapeDtypeStruct((n, c), F32),
    )(q, y, wo, bo)


_CH = 80


def _sc_softmax(st, src, dst):
    n = st.shape[1]
    e = src.shape[0]
    ec = e // NS
    ew = e // NW
    ch = _CH
    rc = ec // ch
    rw = ew // ch
    gpc = ch // L

    @functools.partial(
        pl.kernel,
        out_type=jax.ShapeDtypeStruct((e,), F32),
        mesh=_mesh(),
        compiler_params=pltpu.CompilerParams(needs_layout_passes=False),
        scratch_types=[
            pltpu.VMEM((n,), F32),
            pltpu.VMEM((n,), F32),
            pltpu.VMEM((ec,), I32),
            pltpu.VMEM((ec,), I32),
            pltpu.VMEM((ec,), F32),
            pltpu.VMEM((n,), F32),
            pltpu.VMEM((L,), F32),
            pltpu.VMEM((NS, L), F32),
            pltpu.VMEM((ch,), I32),
            pltpu.VMEM_SHARED((n,), F32),
            pltpu.VMEM_SHARED((NS, L), F32),
        ],
    )
    def k(st_hbm, src_hbm, dst_hbm, alpha_hbm, s_buf, t_buf, srcb, dstb, eb,
          dnm, mbuf, mall, dstb1, dnm_sh, smax_sh):
        cid = lax.axis_index("c")
        sid = lax.axis_index("s")
        eoff = sid * ec
        pltpu.sync_copy(st_hbm.at[0], s_buf)
        pltpu.sync_copy(st_hbm.at[1], t_buf)
        pltpu.sync_copy(src_hbm.at[pl.ds(eoff, ec)], srcb)
        pltpu.sync_copy(dst_hbm.at[pl.ds(eoff, ec)], dstb)

        def zbody(i, _):
            dnm[pl.ds(i * L, L)] = jnp.zeros((L,), F32)
            return 0

        lax.fori_loop(0, n // L, zbody, 0)
        zch = (n // NS) & ~7
        pltpu.sync_copy(dnm.at[pl.ds(sid * zch, zch)],
                        dnm_sh.at[pl.ds(sid * zch, zch)])
        rem = n - NS * zch
        if rem:
            @pl.when(sid == 0)
            def _():
                pltpu.sync_copy(dnm.at[pl.ds(NS * zch, rem)],
                                dnm_sh.at[pl.ds(NS * zch, rem)])

        def ebody(r, m):
            for g in range(gpc):
                o = r * ch + g * L
                sv = plsc.load_gather(s_buf, [srcb[pl.ds(o, L)]])
                tv = plsc.load_gather(t_buf, [dstb[pl.ds(o, L)]])
                ev = sv + tv
                ev = jnp.where(ev >= 0.0, ev, 0.2 * ev)
                eb[pl.ds(o, L)] = ev
                m = jnp.maximum(m, ev)
            return m

        m = lax.fori_loop(0, rc, ebody, jnp.full((L,), -jnp.inf, F32))
        mbuf[...] = m
        pltpu.sync_copy(mbuf, smax_sh.at[sid])
        plsc.subcore_barrier()
        pltpu.sync_copy(smax_sh, mall)
        mm = mall[0]
        for i in range(1, NS):
            mm = jnp.maximum(mm, mall[i])
        gmax = jnp.max(mm)

        def scbody(r, _):
            for g in range(gpc):
                ev = eb[pl.ds(r * ch + g * L, L)]
                eb[pl.ds(r * ch + g * L, L)] = jnp.exp(ev - gmax)
            pltpu.sync_copy(dst_hbm.at[pl.ds(eoff + r * ch, ch)], dstb1)
            pltpu.sync_copy(eb.at[pl.ds(r * ch, ch)], dnm_sh.at[dstb1],
                            add=True)
            return 0

        lax.fori_loop(0, rc, scbody, 0)
        plsc.subcore_barrier()
        pltpu.sync_copy(dnm_sh, dnm)
        roff = cid * rw

        def abody(r, _):
            for g in range(gpc):
                o = (roff + r) * ch + g * L
                ee = eb[pl.ds(o, L)]
                dv = plsc.load_gather(dnm, [dstb[pl.ds(o, L)]])
                eb[pl.ds(o, L)] = ee / (dv + 1e-16)
            return 0

        lax.fori_loop(0, rw, abody, 0)
        pltpu.sync_copy(eb.at[pl.ds(roff * ch, ew)],
                        alpha_hbm.at[pl.ds(eoff + cid * ew, ew)])

    return k(st, src, dst)


def _sc_msg(u, src, dst, alpha):
    n, h = u.shape
    e = src.shape[0]
    ew = e // NW
    npad = -(-n // (NS * 128)) * (NS * 128)
    rpt = npad // NS
    zb = _CH
    ch = _CH
    nch = ew // ch
    hv = h // L

    @functools.partial(
        pl.kernel,
        out_type=jax.ShapeDtypeStruct((NC, npad, h), F32),
        mesh=_mesh(),
        compiler_params=pltpu.CompilerParams(needs_layout_passes=False),
        scratch_types=[
            pltpu.VMEM((ch,), I32),
            pltpu.VMEM((ch,), I32),
            pltpu.VMEM((ch,), F32),
            pltpu.VMEM((ch, h), F32),
            pltpu.VMEM_SHARED((npad, h), F32),
        ],
    )
    def k(u_hbm, src_hbm, dst_hbm, a_hbm, out_hbm, srcb1, dstb1, ab1, rows,
          y_sh):
        cid = lax.axis_index("c")
        sid = lax.axis_index("s")
        eoff = (cid * NS + sid) * ew

        def zbody(i, _):
            rows[i // hv, pl.ds((i % hv) * L, L)] = jnp.zeros((L,), F32)
            return 0

        lax.fori_loop(0, zb * hv, zbody, 0)
        for k2 in range(rpt // zb):
            pltpu.sync_copy(rows, y_sh.at[pl.ds(sid * rpt + k2 * zb, zb)])
        plsc.subcore_barrier()

        def mbody(i, _):
            o = eoff + i * ch
            pltpu.sync_copy(src_hbm.at[pl.ds(o, ch)], srcb1)
            pltpu.sync_copy(a_hbm.at[pl.ds(o, ch)], ab1)
            pltpu.sync_copy(u_hbm.at[srcb1], rows)

            def sbody(g, _2):
                av16 = ab1[pl.ds(g * L, L)]
                for kk in range(L):
                    av = av16[kk]
                    for j in range(hv):
                        sl = pl.ds(j * L, L)
                        rows[g * L + kk, sl] = rows[g * L + kk, sl] * av
                return 0

            lax.fori_loop(0, ch // L, sbody, 0)
            pltpu.sync_copy(dst_hbm.at[pl.ds(o, ch)], dstb1)
            pltpu.sync_copy(rows, y_sh.at[dstb1], add=True)
            return 0

        lax.fori_loop(0, nch, mbody, 0)
        plsc.subcore_barrier()
        for k2 in range(rpt // zb):
            r0 = sid * rpt + k2 * zb
            pltpu.sync_copy(y_sh.at[pl.ds(r0, zb)],
                            out_hbm.at[cid].at[pl.ds(r0, zb)])

    return k(u, src, dst, alpha)


def kernel(x, edge_index, a_src, a_dst, W_p, W_q, W_up0, W_up1, W_up2,
           W_down0, W_down1, W_down2, W_out, b_out):
    src = edge_index[0]
    dst = edge_index[1]
    asd = jnp.stack([a_src, a_dst], axis=0)
    st, p, q, u = _tc_pre(x, asd, W_p, W_q, W_up0)
    alpha = _sc_softmax(st, src, dst)
    ups = [W_up0, W_up1, W_up2]
    downs = [W_down0, W_down1, W_down2]
    out = None
    for i in range(3):
        yp = _sc_msg(u, src, dst, alpha)
        p, v = _tc_mid_a(p, yp, downs[i])
        yq = _sc_msg(v, src, dst, alpha)
        if i < 2:
            p, q, u = _tc_mid_b(q, yq, p, ups[i + 1])
        else:
            out = _tc_fin(q, yq, W_out, b_out.reshape(1, -1))
    return out


# speedup vs baseline: 14.7449x; 2.2625x over previous
import functools

import jax
import jax.numpy as jnp
from jax import lax
from jax.experimental import pallas as pl
from jax.experimental.pallas import tpu as pltpu
from jax.experimental.pallas import tpu_sc as plsc

F32 = jnp.float32
I32 = jnp.int32
NC, NS, L = 2, 16, 16
NW = NC * NS
_HI = lax.Precision.HIGHEST


def _dot(a, b):
    return jnp.dot(a, b, precision=_HI, preferred_element_type=F32)


def _mesh():
    return plsc.VectorSubcoreMesh(
        core_axis_name="c", subcore_axis_name="s", num_cores=NC, num_subcores=NS
    )


_NB = 2000


def _tc_pre(x, asd, wp, wq, wu):
    n, d = x.shape
    h = wp.shape[1]

    def st_body(x_ref, asd_ref, st_ref):
        st_ref[...] = lax.dot_general(
            asd_ref[...], x_ref[...], (((1,), (1,)), ((), ())),
            precision=_HI, preferred_element_type=F32)

    st = pl.pallas_call(
        st_body,
        out_shape=jax.ShapeDtypeStruct((2, n), F32),
    )(x, asd)

    def body(x_ref, asd_ref, wp_ref, wq_ref, wu_ref, p_ref, q_ref, u_ref):
        xb = x_ref[...]
        p_ref[...] = _dot(xb, wp_ref[...])
        qb = _dot(xb, wq_ref[...])
        q_ref[...] = qb
        u_ref[...] = _dot(qb, wu_ref[...])

    p, q, u = pl.pallas_call(
        body,
        grid=(n // _NB,),
        in_specs=[
            pl.BlockSpec((_NB, d), lambda i: (i, 0)),
            pl.BlockSpec((2, d), lambda i: (0, 0)),
            pl.BlockSpec((d, h), lambda i: (0, 0)),
            pl.BlockSpec((d, h), lambda i: (0, 0)),
            pl.BlockSpec((h, h), lambda i: (0, 0)),
        ],
        out_specs=[
            pl.BlockSpec((_NB, h), lambda i: (i, 0)),
            pl.BlockSpec((_NB, h), lambda i: (i, 0)),
            pl.BlockSpec((_NB, h), lambda i: (i, 0)),
        ],
        out_shape=[
            jax.ShapeDtypeStruct((n, h), F32),
            jax.ShapeDtypeStruct((n, h), F32),
            jax.ShapeDtypeStruct((n, h), F32),
        ],
    )(x, asd, wp, wq, wu)
    return st, p, q, u


def _tc_mid_a(p, y, wd):
    n, h = p.shape

    def body(p_ref, y_ref, wd_ref, pn_ref, v_ref):
        pn = p_ref[...] + y_ref[0] + y_ref[1]
        pn_ref[...] = pn
        v_ref[...] = _dot(pn, wd_ref[...])

    return pl.pallas_call(
        body,
        grid=(n // _NB,),
        in_specs=[
            pl.BlockSpec((_NB, h), lambda i: (i, 0)),
            pl.BlockSpec((2, _NB, h), lambda i: (0, i, 0)),
            pl.BlockSpec((h, h), lambda i: (0, 0)),
        ],
        out_specs=[
            pl.BlockSpec((_NB, h), lambda i: (i, 0)),
            pl.BlockSpec((_NB, h), lambda i: (i, 0)),
        ],
        out_shape=[
            jax.ShapeDtypeStruct((n, h), F32),
            jax.ShapeDtypeStruct((n, h), F32),
        ],
    )(p, y, wd)


def _tc_mid_b(q, y, p, wu):
    n, h = q.shape

    def body(q_ref, y_ref, p_ref, wu_ref, pn_ref, qn_ref, u_ref):
        qn = jnp.tanh(q_ref[...] - (y_ref[0] + y_ref[1]))
        pn_ref[...] = jnp.tanh(p_ref[...])
        qn_ref[...] = qn
        u_ref[...] = _dot(qn, wu_ref[...])

    return pl.pallas_call(
        body,
        grid=(n // _NB,),
        in_specs=[
            pl.BlockSpec((_NB, h), lambda i: (i, 0)),
            pl.BlockSpec((2, _NB, h), lambda i: (0, i, 0)),
            pl.BlockSpec((_NB, h), lambda i: (i, 0)),
            pl.BlockSpec((h, h), lambda i: (0, 0)),
        ],
        out_specs=[
            pl.BlockSpec((_NB, h), lambda i: (i, 0)),
            pl.BlockSpec((_NB, h), lambda i: (i, 0)),
            pl.BlockSpec((_NB, h), lambda i: (i, 0)),
        ],
        out_shape=[
            jax.ShapeDtypeStruct((n, h), F32),
            jax.ShapeDtypeStruct((n, h), F32),
            jax.ShapeDtypeStruct((n, h), F32),
        ],
    )(q, y, p, wu)


def _tc_fin(q, y, wo, bo):
    n, h = q.shape
    c = wo.shape[1]

    def body(q_ref, y_ref, wo_ref, bo_ref, o_ref):
        qn = q_ref[...] - (y_ref[0] + y_ref[1])
        o_ref[...] = _dot(qn, wo_ref[...]) + bo_ref[...]

    return pl.pallas_call(
        body,
        grid=(n // _NB,),
        in_specs=[
            pl.BlockSpec((_NB, h), lambda i: (i, 0)),
            pl.BlockSpec((2, _NB, h), lambda i: (0, i, 0)),
            pl.BlockSpec((h, c), lambda i: (0, 0)),
            pl.BlockSpec((1, c), lambda i: (0, 0)),
        ],
        out_specs=pl.BlockSpec((_NB, c), lambda i: (i, 0)),
        out_shape=jax.ShapeDtypeStruct((n, c), F32),
    )(q, y, wo, bo)


_CH = 80


def _sc_softmax(st, src, dst):
    n = st.shape[1]
    e = src.shape[0]
    ec = e // NS
    ew = e // NW
    ch = _CH
    rc = ec // ch
    rw = ew // ch
    gpc = ch // L

    @functools.partial(
        pl.kernel,
        out_type=jax.ShapeDtypeStruct((e,), F32),
        mesh=_mesh(),
        compiler_params=pltpu.CompilerParams(needs_layout_passes=False),
        scratch_types=[
            pltpu.VMEM((n,), F32),
            pltpu.VMEM((n,), F32),
            pltpu.VMEM((ec,), I32),
            pltpu.VMEM((ec,), I32),
            pltpu.VMEM((ec,), F32),
            pltpu.VMEM((n,), F32),
            pltpu.VMEM((L,), F32),
            pltpu.VMEM((NS, L), F32),
            pltpu.VMEM((ch,), I32),
            pltpu.VMEM_SHARED((n,), F32),
            pltpu.VMEM_SHARED((NS, L), F32),
        ],
    )
    def k(st_hbm, src_hbm, dst_hbm, alpha_hbm, s_buf, t_buf, srcb, dstb, eb,
          dnm, mbuf, mall, dstb1, dnm_sh, smax_sh):
        cid = lax.axis_index("c")
        sid = lax.axis_index("s")
        eoff = sid * ec
        pltpu.sync_copy(st_hbm.at[0], s_buf)
        pltpu.sync_copy(st_hbm.at[1], t_buf)
        pltpu.sync_copy(src_hbm.at[pl.ds(eoff, ec)], srcb)
        pltpu.sync_copy(dst_hbm.at[pl.ds(eoff, ec)], dstb)

        def zbody(i, _):
            dnm[pl.ds(i * L, L)] = jnp.zeros((L,), F32)
            return 0

        lax.fori_loop(0, n // L, zbody, 0)
        zch = (n // NS) & ~7
        pltpu.sync_copy(dnm.at[pl.ds(sid * zch, zch)],
                        dnm_sh.at[pl.ds(sid * zch, zch)])
        rem = n - NS * zch
        if rem:
            @pl.when(sid == 0)
            def _():
                pltpu.sync_copy(dnm.at[pl.ds(NS * zch, rem)],
                                dnm_sh.at[pl.ds(NS * zch, rem)])

        def ebody(r, m):
            for g in range(gpc):
                o = r * ch + g * L
                sv = plsc.load_gather(s_buf, [srcb[pl.ds(o, L)]])
                tv = plsc.load_gather(t_buf, [dstb[pl.ds(o, L)]])
                ev = sv + tv
                ev = jnp.where(ev >= 0.0, ev, 0.2 * ev)
                eb[pl.ds(o, L)] = ev
                m = jnp.maximum(m, ev)
            return m

        m = lax.fori_loop(0, rc, ebody, jnp.full((L,), -jnp.inf, F32))
        mbuf[...] = m
        pltpu.sync_copy(mbuf, smax_sh.at[sid])
        plsc.subcore_barrier()
        pltpu.sync_copy(smax_sh, mall)
        mm = mall[0]
        for i in range(1, NS):
            mm = jnp.maximum(mm, mall[i])
        gmax = jnp.max(mm)

        def scbody(r, _):
            for g in range(gpc):
                ev = eb[pl.ds(r * ch + g * L, L)]
                eb[pl.ds(r * ch + g * L, L)] = jnp.exp(ev - gmax)
            pltpu.sync_copy(dst_hbm.at[pl.ds(eoff + r * ch, ch)], dstb1)
            pltpu.sync_copy(eb.at[pl.ds(r * ch, ch)], dnm_sh.at[dstb1],
                            add=True)
            return 0

        lax.fori_loop(0, rc, scbody, 0)
        plsc.subcore_barrier()
        pltpu.sync_copy(dnm_sh, dnm)
        roff = cid * rw

        def abody(r, _):
            for g in range(gpc):
                o = (roff + r) * ch + g * L
                ee = eb[pl.ds(o, L)]
                dv = plsc.load_gather(dnm, [dstb[pl.ds(o, L)]])
                eb[pl.ds(o, L)] = ee / (dv + 1e-16)
            return 0

        lax.fori_loop(0, rw, abody, 0)
        pltpu.sync_copy(eb.at[pl.ds(roff * ch, ew)],
                        alpha_hbm.at[pl.ds(eoff + cid * ew, ew)])

    return k(st, src, dst)


def _sc_msg(u, src, dst, alpha):
    n, h = u.shape
    e = src.shape[0]
    ew = e // NW
    npad = -(-n // (NS * 128)) * (NS * 128)
    rpt = npad // NS
    zb = _CH
    ch = _CH
    nch = ew // ch
    hv = h // L

    sch = 25
    nsup = nch // sch

    @functools.partial(
        pl.kernel,
        out_type=jax.ShapeDtypeStruct((NC, npad, h), F32),
        mesh=_mesh(),
        compiler_params=pltpu.CompilerParams(needs_layout_passes=False),
        scratch_types=[
            pltpu.VMEM((sch * ch,), I32),
            pltpu.VMEM((sch * ch,), F32),
            pltpu.VMEM((ch,), I32),
            pltpu.VMEM((ch,), I32),
            pltpu.VMEM((ch, h), F32),
            pltpu.VMEM((ch, h), F32),
            pltpu.SemaphoreType.DMA,
            pltpu.SemaphoreType.DMA,
            pltpu.SemaphoreType.DMA,
            pltpu.SemaphoreType.DMA,
            pltpu.VMEM_SHARED((npad, h), F32),
        ],
    )
    def k(u_hbm, src_hbm, dst_hbm, a_hbm, out_hbm, srcb, ab, dstA, dstB,
          rowsA, rowsB, gsA, gsB, ssA, ssB, y_sh):
        cid = lax.axis_index("c")
        sid = lax.axis_index("s")
        eoff = (cid * NS + sid) * ew

        def zbody(i, _):
            rowsA[i // hv, pl.ds((i % hv) * L, L)] = jnp.zeros((L,), F32)
            return 0

        lax.fori_loop(0, zb * hv, zbody, 0)
        for k2 in range(rpt // zb):
            pltpu.sync_copy(rowsA, y_sh.at[pl.ds(sid * rpt + k2 * zb, zb)])
        plsc.subcore_barrier()

        for sup in range(nsup):
            o0 = eoff + sup * sch * ch

            def fire(j, dstb, rows, gs):
                pltpu.async_copy(dst_hbm.at[pl.ds(o0 + j * ch, ch)], dstb, gs)
                pltpu.async_copy(u_hbm.at[srcb.at[pl.ds(j * ch, ch)]], rows,
                                 gs)

            def proc(j, dstb, rows, gs, ss):
                pltpu.make_async_copy(
                    dst_hbm.at[pl.ds(o0 + j * ch, ch)], dstb, gs).wait()
                pltpu.make_async_copy(
                    u_hbm.at[srcb.at[pl.ds(j * ch, ch)]], rows, gs).wait()

                def sbody(g, _2):
                    av16 = ab[pl.ds(j * ch + g * L, L)]
                    for kk in range(L):
                        av = av16[kk]
                        for jj in range(hv):
                            sl = pl.ds(jj * L, L)
                            rows[g * L + kk, sl] = rows[g * L + kk, sl] * av
                    return 0

                lax.fori_loop(0, ch // L, sbody, 0)
                pltpu.async_copy(rows, y_sh.at[dstb], ss, add=True)
                pltpu.make_async_copy(rows, y_sh.at[dstb], ss).wait()

            pltpu.sync_copy(src_hbm.at[pl.ds(o0, sch * ch)], srcb)
            pltpu.sync_copy(a_hbm.at[pl.ds(o0, sch * ch)], ab)
            fire(0, dstA, rowsA, gsA)
            fire(1, dstB, rowsB, gsB)

            def pbody(j, _):
                @pl.when(j % 2 == 0)
                def _():
                    proc(j, dstA, rowsA, gsA, ssA)

                    @pl.when(j + 2 < sch)
                    def _():
                        fire(j + 2, dstA, rowsA, gsA)

                @pl.when(j % 2 == 1)
                def _():
                    proc(j, dstB, rowsB, gsB, ssB)

                    @pl.when(j + 2 < sch)
                    def _():
                        fire(j + 2, dstB, rowsB, gsB)

                return 0

            lax.fori_loop(0, sch, pbody, 0)
        plsc.subcore_barrier()
        pltpu.sync_copy(y_sh.at[pl.ds(sid * rpt, rpt)],
                        out_hbm.at[cid].at[pl.ds(sid * rpt, rpt)])

    return k(u, src, dst, alpha)


def kernel(x, edge_index, a_src, a_dst, W_p, W_q, W_up0, W_up1, W_up2,
           W_down0, W_down1, W_down2, W_out, b_out):
    src = edge_index[0]
    dst = edge_index[1]
    asd = jnp.stack([a_src, a_dst], axis=0)
    st, p, q, u = _tc_pre(x, asd, W_p, W_q, W_up0)
    alpha = _sc_softmax(st, src, dst)
    ups = [W_up0, W_up1, W_up2]
    downs = [W_down0, W_down1, W_down2]
    out = None
    for i in range(3):
        yp = _sc_msg(u, src, dst, alpha)
        p, v = _tc_mid_a(p, yp, downs[i])
        yq = _sc_msg(v, src, dst, alpha)
        if i < 2:
            p, q, u = _tc_mid_b(q, yq, p, ups[i + 1])
        else:
            out = _tc_fin(q, yq, W_out, b_out.reshape(1, -1))
    return out


# speedup vs baseline: 16.0367x; 1.0876x over previous
import functools

import jax
import jax.numpy as jnp
from jax import lax
from jax.experimental import pallas as pl
from jax.experimental.pallas import tpu as pltpu
from jax.experimental.pallas import tpu_sc as plsc

F32 = jnp.float32
I32 = jnp.int32
NC, NS, L = 2, 16, 16
NW = NC * NS
_HI = lax.Precision.HIGHEST


def _dot(a, b):
    return jnp.dot(a, b, precision=_HI, preferred_element_type=F32)


def _mesh():
    return plsc.VectorSubcoreMesh(
        core_axis_name="c", subcore_axis_name="s", num_cores=NC, num_subcores=NS
    )


_NB = 2000


def _tc_pre(x, asd, wp, wq, wu):
    n, d = x.shape
    h = wp.shape[1]

    def st_body(x_ref, asd_ref, st_ref):
        st_ref[...] = lax.dot_general(
            asd_ref[...], x_ref[...], (((1,), (1,)), ((), ())),
            precision=_HI, preferred_element_type=F32)

    st = pl.pallas_call(
        st_body,
        out_shape=jax.ShapeDtypeStruct((2, n), F32),
    )(x, asd)

    def body(x_ref, asd_ref, wp_ref, wq_ref, wu_ref, p_ref, q_ref, u_ref):
        xb = x_ref[...]
        p_ref[...] = _dot(xb, wp_ref[...])
        qb = _dot(xb, wq_ref[...])
        q_ref[...] = qb
        u_ref[...] = _dot(qb, wu_ref[...])

    p, q, u = pl.pallas_call(
        body,
        grid=(n // _NB,),
        in_specs=[
            pl.BlockSpec((_NB, d), lambda i: (i, 0)),
            pl.BlockSpec((2, d), lambda i: (0, 0)),
            pl.BlockSpec((d, h), lambda i: (0, 0)),
            pl.BlockSpec((d, h), lambda i: (0, 0)),
            pl.BlockSpec((h, h), lambda i: (0, 0)),
        ],
        out_specs=[
            pl.BlockSpec((_NB, h), lambda i: (i, 0)),
            pl.BlockSpec((_NB, h), lambda i: (i, 0)),
            pl.BlockSpec((_NB, h), lambda i: (i, 0)),
        ],
        out_shape=[
            jax.ShapeDtypeStruct((n, h), F32),
            jax.ShapeDtypeStruct((n, h), F32),
            jax.ShapeDtypeStruct((n, h), F32),
        ],
    )(x, asd, wp, wq, wu)
    return st, p, q, u


def _tc_mid_a(p, y, wd):
    n, h = p.shape

    def body(p_ref, y_ref, wd_ref, pn_ref, v_ref):
        pn = p_ref[...] + y_ref[0] + y_ref[1]
        pn_ref[...] = pn
        v_ref[...] = _dot(pn, wd_ref[...])

    return pl.pallas_call(
        body,
        grid=(n // _NB,),
        in_specs=[
            pl.BlockSpec((_NB, h), lambda i: (i, 0)),
            pl.BlockSpec((2, _NB, h), lambda i: (0, i, 0)),
            pl.BlockSpec((h, h), lambda i: (0, 0)),
        ],
        out_specs=[
            pl.BlockSpec((_NB, h), lambda i: (i, 0)),
            pl.BlockSpec((_NB, h), lambda i: (i, 0)),
        ],
        out_shape=[
            jax.ShapeDtypeStruct((n, h), F32),
            jax.ShapeDtypeStruct((n, h), F32),
        ],
    )(p, y, wd)


def _tc_mid_b(q, y, p, wu):
    n, h = q.shape

    def body(q_ref, y_ref, p_ref, wu_ref, pn_ref, qn_ref, u_ref):
        qn = jnp.tanh(q_ref[...] - (y_ref[0] + y_ref[1]))
        pn_ref[...] = jnp.tanh(p_ref[...])
        qn_ref[...] = qn
        u_ref[...] = _dot(qn, wu_ref[...])

    return pl.pallas_call(
        body,
        grid=(n // _NB,),
        in_specs=[
            pl.BlockSpec((_NB, h), lambda i: (i, 0)),
            pl.BlockSpec((2, _NB, h), lambda i: (0, i, 0)),
            pl.BlockSpec((_NB, h), lambda i: (i, 0)),
            pl.BlockSpec((h, h), lambda i: (0, 0)),
        ],
        out_specs=[
            pl.BlockSpec((_NB, h), lambda i: (i, 0)),
            pl.BlockSpec((_NB, h), lambda i: (i, 0)),
            pl.BlockSpec((_NB, h), lambda i: (i, 0)),
        ],
        out_shape=[
            jax.ShapeDtypeStruct((n, h), F32),
            jax.ShapeDtypeStruct((n, h), F32),
            jax.ShapeDtypeStruct((n, h), F32),
        ],
    )(q, y, p, wu)


def _tc_fin(q, y, wo, bo):
    n, h = q.shape
    c = wo.shape[1]

    def body(q_ref, y_ref, wo_ref, bo_ref, o_ref):
        qn = q_ref[...] - (y_ref[0] + y_ref[1])
        o_ref[...] = _dot(qn, wo_ref[...]) + bo_ref[...]

    return pl.pallas_call(
        body,
        grid=(n // _NB,),
        in_specs=[
            pl.BlockSpec((_NB, h), lambda i: (i, 0)),
            pl.BlockSpec((2, _NB, h), lambda i: (0, i, 0)),
            pl.BlockSpec((h, c), lambda i: (0, 0)),
            pl.BlockSpec((1, c), lambda i: (0, 0)),
        ],
        out_specs=pl.BlockSpec((_NB, c), lambda i: (i, 0)),
        out_shape=jax.ShapeDtypeStruct((n, c), F32),
    )(q, y, wo, bo)


_CH = 80


def _sc_softmax(st, src, dst):
    n = st.shape[1]
    e = src.shape[0]
    ec = e // NS
    ew = e // NW
    ch = _CH
    rc = ec // ch
    rw = ew // ch
    gpc = ch // L

    @functools.partial(
        pl.kernel,
        out_type=jax.ShapeDtypeStruct((e,), F32),
        mesh=_mesh(),
        compiler_params=pltpu.CompilerParams(needs_layout_passes=False),
        scratch_types=[
            pltpu.VMEM((n,), F32),
            pltpu.VMEM((n,), F32),
            pltpu.VMEM((ec,), I32),
            pltpu.VMEM((ec,), I32),
            pltpu.VMEM((ec,), F32),
            pltpu.VMEM((n,), F32),
            pltpu.VMEM((L,), F32),
            pltpu.VMEM((NS, L), F32),
            pltpu.SemaphoreType.DMA,
            pltpu.VMEM_SHARED((n,), F32),
            pltpu.VMEM_SHARED((NS, L), F32),
        ],
    )
    def k(st_hbm, src_hbm, dst_hbm, alpha_hbm, s_buf, t_buf, srcb, dstb, eb,
          dnm, mbuf, mall, ssem, dnm_sh, smax_sh):
        cid = lax.axis_index("c")
        sid = lax.axis_index("s")
        eoff = sid * ec
        pltpu.sync_copy(st_hbm.at[0], s_buf)
        pltpu.sync_copy(st_hbm.at[1], t_buf)
        pltpu.sync_copy(src_hbm.at[pl.ds(eoff, ec)], srcb)
        pltpu.sync_copy(dst_hbm.at[pl.ds(eoff, ec)], dstb)

        def zbody(i, _):
            dnm[pl.ds(i * L, L)] = jnp.zeros((L,), F32)
            return 0

        lax.fori_loop(0, n // L, zbody, 0)
        zch = (n // NS) & ~7
        pltpu.sync_copy(dnm.at[pl.ds(sid * zch, zch)],
                        dnm_sh.at[pl.ds(sid * zch, zch)])
        rem = n - NS * zch
        if rem:
            @pl.when(sid == 0)
            def _():
                pltpu.sync_copy(dnm.at[pl.ds(NS * zch, rem)],
                                dnm_sh.at[pl.ds(NS * zch, rem)])

        def ebody(r, m):
            for g in range(gpc):
                o = r * ch + g * L
                sv = plsc.load_gather(s_buf, [srcb[pl.ds(o, L)]])
                tv = plsc.load_gather(t_buf, [dstb[pl.ds(o, L)]])
                ev = sv + tv
                ev = jnp.where(ev >= 0.0, ev, 0.2 * ev)
                eb[pl.ds(o, L)] = ev
                m = jnp.maximum(m, ev)
            return m

        m = lax.fori_loop(0, rc, ebody, jnp.full((L,), -jnp.inf, F32))
        mbuf[...] = m
        pltpu.sync_copy(mbuf, smax_sh.at[sid])
        plsc.subcore_barrier()
        pltpu.sync_copy(smax_sh, mall)
        mm = mall[0]
        for i in range(1, NS):
            mm = jnp.maximum(mm, mall[i])
        gmax = jnp.max(mm)

        def expbody(r, _):
            for g in range(gpc):
                ev = eb[pl.ds(r * ch + g * L, L)]
                eb[pl.ds(r * ch + g * L, L)] = jnp.exp(ev - gmax)
            return 0

        lax.fori_loop(0, rc, expbody, 0)
        bat = 25
        for b0 in range(rc // bat):
            for j in range(bat):
                r = b0 * bat + j
                pltpu.async_copy(eb.at[pl.ds(r * ch, ch)],
                                 dnm_sh.at[dstb.at[pl.ds(r * ch, ch)]],
                                 ssem, add=True)
            for j in range(bat):
                r = b0 * bat + j
                pltpu.make_async_copy(eb.at[pl.ds(r * ch, ch)],
                                      dnm_sh.at[dstb.at[pl.ds(r * ch, ch)]],
                                      ssem).wait()
        plsc.subcore_barrier()
        pltpu.sync_copy(dnm_sh, dnm)
        roff = cid * rw

        def abody(r, _):
            for g in range(gpc):
                o = (roff + r) * ch + g * L
                ee = eb[pl.ds(o, L)]
                dv = plsc.load_gather(dnm, [dstb[pl.ds(o, L)]])
                eb[pl.ds(o, L)] = ee / (dv + 1e-16)
            return 0

        lax.fori_loop(0, rw, abody, 0)
        pltpu.sync_copy(eb.at[pl.ds(roff * ch, ew)],
                        alpha_hbm.at[pl.ds(eoff + cid * ew, ew)])

    return k(st, src, dst)


def _sc_msg(u, src, dst, alpha):
    n, h = u.shape
    e = src.shape[0]
    ew = e // NW
    npad = -(-n // (NS * 128)) * (NS * 128)
    rpt = npad // NS
    zb = _CH
    ch = _CH
    nch = ew // ch
    hv = h // L

    sch = 25
    nsup = nch // sch

    @functools.partial(
        pl.kernel,
        out_type=jax.ShapeDtypeStruct((NC, npad, h), F32),
        mesh=_mesh(),
        compiler_params=pltpu.CompilerParams(needs_layout_passes=False),
        scratch_types=[
            pltpu.VMEM((sch * ch,), I32),
            pltpu.VMEM((sch * ch,), I32),
            pltpu.VMEM((sch * ch,), F32),
            pltpu.VMEM((ch, h), F32),
            pltpu.VMEM((ch, h), F32),
            pltpu.SemaphoreType.DMA,
            pltpu.SemaphoreType.DMA,
            pltpu.SemaphoreType.DMA,
            pltpu.SemaphoreType.DMA,
            pltpu.VMEM_SHARED((npad, h), F32),
        ],
    )
    def k(u_hbm, src_hbm, dst_hbm, a_hbm, out_hbm, srcb, dstb, ab,
          rowsA, rowsB, gsA, gsB, ssA, ssB, y_sh):
        cid = lax.axis_index("c")
        sid = lax.axis_index("s")
        eoff = (cid * NS + sid) * ew

        def zbody(i, _):
            rowsA[i // hv, pl.ds((i % hv) * L, L)] = jnp.zeros((L,), F32)
            return 0

        lax.fori_loop(0, zb * hv, zbody, 0)
        for k2 in range(rpt // zb):
            pltpu.sync_copy(rowsA, y_sh.at[pl.ds(sid * rpt + k2 * zb, zb)])
        plsc.subcore_barrier()

        for sup in range(nsup):
            o0 = eoff + sup * sch * ch

            def fire(j, rows, gs):
                pltpu.async_copy(u_hbm.at[srcb.at[pl.ds(j * ch, ch)]], rows,
                                 gs)

            def proc(j, rows, gs, ss):
                pltpu.make_async_copy(
                    u_hbm.at[srcb.at[pl.ds(j * ch, ch)]], rows, gs).wait()

                def sbody(g, _2):
                    av16 = ab[pl.ds(j * ch + g * L, L)]
                    for kk in range(L):
                        av = av16[kk]
                        for jj in range(hv):
                            sl = pl.ds(jj * L, L)
                            rows[g * L + kk, sl] = rows[g * L + kk, sl] * av
                    return 0

                lax.fori_loop(0, ch // L, sbody, 0)
                ysl = y_sh.at[dstb.at[pl.ds(j * ch, ch)]]
                pltpu.async_copy(rows, ysl, ss, add=True)
                pltpu.make_async_copy(rows, ysl, ss).wait()

            pltpu.sync_copy(src_hbm.at[pl.ds(o0, sch * ch)], srcb)
            pltpu.sync_copy(dst_hbm.at[pl.ds(o0, sch * ch)], dstb)
            pltpu.sync_copy(a_hbm.at[pl.ds(o0, sch * ch)], ab)
            fire(0, rowsA, gsA)
            fire(1, rowsB, gsB)

            def pbody(j, _):
                @pl.when(j % 2 == 0)
                def _():
                    proc(j, rowsA, gsA, ssA)

                    @pl.when(j + 2 < sch)
                    def _():
                        fire(j + 2, rowsA, gsA)

                @pl.when(j % 2 == 1)
                def _():
                    proc(j, rowsB, gsB, ssB)

                    @pl.when(j + 2 < sch)
                    def _():
                        fire(j + 2, rowsB, gsB)

                return 0

            lax.fori_loop(0, sch, pbody, 0)
        plsc.subcore_barrier()
        pltpu.sync_copy(y_sh.at[pl.ds(sid * rpt, rpt)],
                        out_hbm.at[cid].at[pl.ds(sid * rpt, rpt)])

    return k(u, src, dst, alpha)


def kernel(x, edge_index, a_src, a_dst, W_p, W_q, W_up0, W_up1, W_up2,
           W_down0, W_down1, W_down2, W_out, b_out):
    src = edge_index[0]
    dst = edge_index[1]
    asd = jnp.stack([a_src, a_dst], axis=0)
    st, p, q, u = _tc_pre(x, asd, W_p, W_q, W_up0)
    alpha = _sc_softmax(st, src, dst)
    ups = [W_up0, W_up1, W_up2]
    downs = [W_down0, W_down1, W_down2]
    out = None
    for i in range(3):
        yp = _sc_msg(u, src, dst, alpha)
        p, v = _tc_mid_a(p, yp, downs[i])
        yq = _sc_msg(v, src, dst, alpha)
        if i < 2:
            p, q, u = _tc_mid_b(q, yq, p, ups[i + 1])
        else:
            out = _tc_fin(q, yq, W_out, b_out.reshape(1, -1))
    return out


# speedup vs baseline: 17.5136x; 1.0921x over previous
import functools

import jax
import jax.numpy as jnp
from jax import lax
from jax.experimental import pallas as pl
from jax.experimental.pallas import tpu as pltpu
from jax.experimental.pallas import tpu_sc as plsc

F32 = jnp.float32
I32 = jnp.int32
NC, NS, L = 2, 16, 16
NW = NC * NS
_HI = lax.Precision.HIGHEST


def _dot(a, b):
    return jnp.dot(a, b, precision=_HI, preferred_element_type=F32)


def _mesh():
    return plsc.VectorSubcoreMesh(
        core_axis_name="c", subcore_axis_name="s", num_cores=NC, num_subcores=NS
    )


_NB = 2000


def _tc_pre(x, asd, wp, wq, wu):
    n, d = x.shape
    h = wp.shape[1]

    def st_body(x_ref, asd_ref, st_ref):
        st_ref[...] = lax.dot_general(
            asd_ref[...], x_ref[...], (((1,), (1,)), ((), ())),
            precision=_HI, preferred_element_type=F32)

    st = pl.pallas_call(
        st_body,
        out_shape=jax.ShapeDtypeStruct((2, n), F32),
    )(x, asd)

    def body(x_ref, asd_ref, wp_ref, wq_ref, wu_ref, p_ref, q_ref, u_ref):
        xb = x_ref[...]
        p_ref[...] = _dot(xb, wp_ref[...])
        qb = _dot(xb, wq_ref[...])
        q_ref[...] = qb
        u_ref[...] = _dot(qb, wu_ref[...])

    p, q, u = pl.pallas_call(
        body,
        grid=(n // _NB,),
        in_specs=[
            pl.BlockSpec((_NB, d), lambda i: (i, 0)),
            pl.BlockSpec((2, d), lambda i: (0, 0)),
            pl.BlockSpec((d, h), lambda i: (0, 0)),
            pl.BlockSpec((d, h), lambda i: (0, 0)),
            pl.BlockSpec((h, h), lambda i: (0, 0)),
        ],
        out_specs=[
            pl.BlockSpec((_NB, h), lambda i: (i, 0)),
            pl.BlockSpec((_NB, h), lambda i: (i, 0)),
            pl.BlockSpec((_NB, h), lambda i: (i, 0)),
        ],
        out_shape=[
            jax.ShapeDtypeStruct((n, h), F32),
            jax.ShapeDtypeStruct((n, h), F32),
            jax.ShapeDtypeStruct((n, h), F32),
        ],
    )(x, asd, wp, wq, wu)
    return st, p, q, u


def _tc_mid_a(p, y, wd):
    n, h = p.shape

    def body(p_ref, y_ref, wd_ref, pn_ref, v_ref):
        pn = p_ref[...] + y_ref[0] + y_ref[1]
        pn_ref[...] = pn
        v_ref[...] = _dot(pn, wd_ref[...])

    return pl.pallas_call(
        body,
        grid=(n // _NB,),
        in_specs=[
            pl.BlockSpec((_NB, h), lambda i: (i, 0)),
            pl.BlockSpec((2, _NB, h), lambda i: (0, i, 0)),
            pl.BlockSpec((h, h), lambda i: (0, 0)),
        ],
        out_specs=[
            pl.BlockSpec((_NB, h), lambda i: (i, 0)),
            pl.BlockSpec((_NB, h), lambda i: (i, 0)),
        ],
        out_shape=[
            jax.ShapeDtypeStruct((n, h), F32),
            jax.ShapeDtypeStruct((n, h), F32),
        ],
    )(p, y, wd)


def _tc_mid_b(q, y, p, wu):
    n, h = q.shape

    def body(q_ref, y_ref, p_ref, wu_ref, pn_ref, qn_ref, u_ref):
        qn = jnp.tanh(q_ref[...] - (y_ref[0] + y_ref[1]))
        pn_ref[...] = jnp.tanh(p_ref[...])
        qn_ref[...] = qn
        u_ref[...] = _dot(qn, wu_ref[...])

    return pl.pallas_call(
        body,
        grid=(n // _NB,),
        in_specs=[
            pl.BlockSpec((_NB, h), lambda i: (i, 0)),
            pl.BlockSpec((2, _NB, h), lambda i: (0, i, 0)),
            pl.BlockSpec((_NB, h), lambda i: (i, 0)),
            pl.BlockSpec((h, h), lambda i: (0, 0)),
        ],
        out_specs=[
            pl.BlockSpec((_NB, h), lambda i: (i, 0)),
            pl.BlockSpec((_NB, h), lambda i: (i, 0)),
            pl.BlockSpec((_NB, h), lambda i: (i, 0)),
        ],
        out_shape=[
            jax.ShapeDtypeStruct((n, h), F32),
            jax.ShapeDtypeStruct((n, h), F32),
            jax.ShapeDtypeStruct((n, h), F32),
        ],
    )(q, y, p, wu)


def _tc_fin(q, y, wo, bo):
    n, h = q.shape
    c = wo.shape[1]

    def body(q_ref, y_ref, wo_ref, bo_ref, o_ref):
        qn = q_ref[...] - (y_ref[0] + y_ref[1])
        o_ref[...] = _dot(qn, wo_ref[...]) + bo_ref[...]

    return pl.pallas_call(
        body,
        grid=(n // _NB,),
        in_specs=[
            pl.BlockSpec((_NB, h), lambda i: (i, 0)),
            pl.BlockSpec((2, _NB, h), lambda i: (0, i, 0)),
            pl.BlockSpec((h, c), lambda i: (0, 0)),
            pl.BlockSpec((1, c), lambda i: (0, 0)),
        ],
        out_specs=pl.BlockSpec((_NB, c), lambda i: (i, 0)),
        out_shape=jax.ShapeDtypeStruct((n, c), F32),
    )(q, y, wo, bo)


_CH = 80


def _sc_softmax(st, src, dst):
    n = st.shape[1]
    e = src.shape[0]
    ec = e // NS
    ew = e // NW
    ch = _CH
    rc = ec // ch
    rw = ew // ch
    gpc = ch // L

    @functools.partial(
        pl.kernel,
        out_type=jax.ShapeDtypeStruct((e,), F32),
        mesh=_mesh(),
        compiler_params=pltpu.CompilerParams(needs_layout_passes=False),
        scratch_types=[
            pltpu.VMEM((n,), F32),
            pltpu.VMEM((n,), F32),
            pltpu.VMEM((ec,), I32),
            pltpu.VMEM((ec,), I32),
            pltpu.VMEM((ec,), F32),
            pltpu.VMEM((n,), F32),
            pltpu.VMEM((L,), F32),
            pltpu.VMEM((NS, L), F32),
            pltpu.SemaphoreType.DMA,
            pltpu.VMEM_SHARED((n,), F32),
            pltpu.VMEM_SHARED((NS, L), F32),
        ],
    )
    def k(st_hbm, src_hbm, dst_hbm, alpha_hbm, s_buf, t_buf, srcb, dstb, eb,
          dnm, mbuf, mall, ssem, dnm_sh, smax_sh):
        cid = lax.axis_index("c")
        sid = lax.axis_index("s")
        eoff = sid * ec
        pltpu.sync_copy(st_hbm.at[0], s_buf)
        pltpu.sync_copy(st_hbm.at[1], t_buf)
        pltpu.sync_copy(src_hbm.at[pl.ds(eoff, ec)], srcb)
        pltpu.sync_copy(dst_hbm.at[pl.ds(eoff, ec)], dstb)

        def zbody(i, _):
            dnm[pl.ds(i * L, L)] = jnp.zeros((L,), F32)
            return 0

        lax.fori_loop(0, n // L, zbody, 0)
        zch = (n // NS) & ~7
        pltpu.sync_copy(dnm.at[pl.ds(sid * zch, zch)],
                        dnm_sh.at[pl.ds(sid * zch, zch)])
        rem = n - NS * zch
        if rem:
            @pl.when(sid == 0)
            def _():
                pltpu.sync_copy(dnm.at[pl.ds(NS * zch, rem)],
                                dnm_sh.at[pl.ds(NS * zch, rem)])

        def ebody(r, m):
            for g in range(gpc):
                o = r * ch + g * L
                sv = plsc.load_gather(s_buf, [srcb[pl.ds(o, L)]])
                tv = plsc.load_gather(t_buf, [dstb[pl.ds(o, L)]])
                ev = sv + tv
                ev = jnp.where(ev >= 0.0, ev, 0.2 * ev)
                eb[pl.ds(o, L)] = ev
                m = jnp.maximum(m, ev)
            return m

        m = lax.fori_loop(0, rc, ebody, jnp.full((L,), -jnp.inf, F32))
        mbuf[...] = m
        pltpu.sync_copy(mbuf, smax_sh.at[sid])
        plsc.subcore_barrier()
        pltpu.sync_copy(smax_sh, mall)
        mm = mall[0]
        for i in range(1, NS):
            mm = jnp.maximum(mm, mall[i])
        gmax = jnp.max(mm)

        def expbody(r, _):
            for g in range(gpc):
                ev = eb[pl.ds(r * ch + g * L, L)]
                eb[pl.ds(r * ch + g * L, L)] = jnp.exp(ev - gmax)
            return 0

        lax.fori_loop(0, rc, expbody, 0)
        bat = 25
        for b0 in range(rc // bat):
            for j in range(bat):
                r = b0 * bat + j
                pltpu.async_copy(eb.at[pl.ds(r * ch, ch)],
                                 dnm_sh.at[dstb.at[pl.ds(r * ch, ch)]],
                                 ssem, add=True)
            for j in range(bat):
                r = b0 * bat + j
                pltpu.make_async_copy(eb.at[pl.ds(r * ch, ch)],
                                      dnm_sh.at[dstb.at[pl.ds(r * ch, ch)]],
                                      ssem).wait()
        plsc.subcore_barrier()
        pltpu.sync_copy(dnm_sh, dnm)
        roff = cid * rw

        def abody(r, _):
            for g in range(gpc):
                o = (roff + r) * ch + g * L
                ee = eb[pl.ds(o, L)]
                dv = plsc.load_gather(dnm, [dstb[pl.ds(o, L)]])
                eb[pl.ds(o, L)] = ee / (dv + 1e-16)
            return 0

        lax.fori_loop(0, rw, abody, 0)
        pltpu.sync_copy(eb.at[pl.ds(roff * ch, ew)],
                        alpha_hbm.at[pl.ds(eoff + cid * ew, ew)])

    return k(st, src, dst)


def _sc_msg(u, src, dst, alpha):
    n, h = u.shape
    e = src.shape[0]
    ew = e // NW
    npad = -(-n // (NS * 128)) * (NS * 128)
    rpt = npad // NS
    zb = _CH
    ch = _CH
    nch = ew // ch
    hv = h // L

    sch = 25
    nsup = nch // sch

    @functools.partial(
        pl.kernel,
        out_type=jax.ShapeDtypeStruct((NC, npad, h), F32),
        mesh=_mesh(),
        compiler_params=pltpu.CompilerParams(needs_layout_passes=False),
        scratch_types=[
            pltpu.VMEM((sch * ch,), I32),
            pltpu.VMEM((sch * ch,), I32),
            pltpu.VMEM((sch * ch,), F32),
            pltpu.VMEM((ch, h), F32),
            pltpu.VMEM((ch, h), F32),
            pltpu.VMEM((ch, h), F32),
            pltpu.SemaphoreType.DMA,
            pltpu.SemaphoreType.DMA,
            pltpu.SemaphoreType.DMA,
            pltpu.SemaphoreType.DMA,
            pltpu.SemaphoreType.DMA,
            pltpu.SemaphoreType.DMA,
            pltpu.VMEM_SHARED((npad, h), F32),
        ],
    )
    def k(u_hbm, src_hbm, dst_hbm, a_hbm, out_hbm, srcb, dstb, ab,
          rowsA, rowsB, rowsC, gsA, gsB, gsC, ssA, ssB, ssC, y_sh):
        cid = lax.axis_index("c")
        sid = lax.axis_index("s")
        eoff = (cid * NS + sid) * ew

        def zbody(i, _):
            rowsA[i // hv, pl.ds((i % hv) * L, L)] = jnp.zeros((L,), F32)
            return 0

        lax.fori_loop(0, zb * hv, zbody, 0)
        for k2 in range(rpt // zb):
            pltpu.sync_copy(rowsA, y_sh.at[pl.ds(sid * rpt + k2 * zb, zb)])
        plsc.subcore_barrier()

        for sup in range(nsup):
            o0 = eoff + sup * sch * ch

            def fire(j, rows, gs):
                pltpu.async_copy(u_hbm.at[srcb.at[pl.ds(j * ch, ch)]], rows,
                                 gs)

            def swait(j, rows, ss):
                pltpu.make_async_copy(
                    rows, y_sh.at[dstb.at[pl.ds(j * ch, ch)]], ss).wait()

            def proc(j, rows, gs, ss, nrows, ngs, nss):
                pltpu.make_async_copy(
                    u_hbm.at[srcb.at[pl.ds(j * ch, ch)]], rows, gs).wait()

                def sbody(g, _2):
                    av16 = ab[pl.ds(j * ch + g * L, L)]
                    for kk in range(L):
                        av = av16[kk]
                        for jj in range(hv):
                            sl = pl.ds(jj * L, L)
                            rows[g * L + kk, sl] = rows[g * L + kk, sl] * av
                    return 0

                lax.fori_loop(0, ch // L, sbody, 0)
                pltpu.async_copy(rows, y_sh.at[dstb.at[pl.ds(j * ch, ch)]],
                                 ss, add=True)

                @pl.when(j + 2 < sch)
                def _():
                    @pl.when(j >= 1)
                    def _():
                        swait(j - 1, nrows, nss)

                    fire(j + 2, nrows, ngs)

            pltpu.sync_copy(src_hbm.at[pl.ds(o0, sch * ch)], srcb)
            pltpu.sync_copy(dst_hbm.at[pl.ds(o0, sch * ch)], dstb)
            pltpu.sync_copy(a_hbm.at[pl.ds(o0, sch * ch)], ab)
            fire(0, rowsA, gsA)
            fire(1, rowsB, gsB)
            bufs = [(rowsA, gsA, ssA), (rowsB, gsB, ssB), (rowsC, gsC, ssC)]

            def pbody(j, _):
                for par in range(3):
                    @pl.when(j % 3 == par)
                    def _(par=par):
                        rows, gs, ss = bufs[par]
                        nrows, ngs, nss = bufs[(par + 2) % 3]
                        proc(j, rows, gs, ss, nrows, ngs, nss)

                return 0

            lax.fori_loop(0, sch, pbody, 0)
            for jj in range(sch - 3, sch):
                rows, gs, ss = bufs[jj % 3]
                swait(jj, rows, ss)
        plsc.subcore_barrier()
        pltpu.sync_copy(y_sh.at[pl.ds(sid * rpt, rpt)],
                        out_hbm.at[cid].at[pl.ds(sid * rpt, rpt)])

    return k(u, src, dst, alpha)


def kernel(x, edge_index, a_src, a_dst, W_p, W_q, W_up0, W_up1, W_up2,
           W_down0, W_down1, W_down2, W_out, b_out):
    src = edge_index[0]
    dst = edge_index[1]
    asd = jnp.stack([a_src, a_dst], axis=0)
    st, p, q, u = _tc_pre(x, asd, W_p, W_q, W_up0)
    alpha = _sc_softmax(st, src, dst)
    ups = [W_up0, W_up1, W_up2]
    downs = [W_down0, W_down1, W_down2]
    out = None
    for i in range(3):
        yp = _sc_msg(u, src, dst, alpha)
        p, v = _tc_mid_a(p, yp, downs[i])
        yq = _sc_msg(v, src, dst, alpha)
        if i < 2:
            p, q, u = _tc_mid_b(q, yq, p, ups[i + 1])
        else:
            out = _tc_fin(q, yq, W_out, b_out.reshape(1, -1))
    return out


# speedup vs baseline: 17.8837x; 1.0211x over previous
import functools

import jax
import jax.numpy as jnp
from jax import lax
from jax.experimental import pallas as pl
from jax.experimental.pallas import tpu as pltpu
from jax.experimental.pallas import tpu_sc as plsc

F32 = jnp.float32
I32 = jnp.int32
NC, NS, L = 2, 16, 16
NW = NC * NS
_HI = lax.Precision.HIGHEST


def _dot(a, b):
    return jnp.dot(a, b, precision=_HI, preferred_element_type=F32)


def _mesh():
    return plsc.VectorSubcoreMesh(
        core_axis_name="c", subcore_axis_name="s", num_cores=NC, num_subcores=NS
    )


_NB = 2000


def _tc_pre(x, asd, wp, wq, wu):
    n, d = x.shape
    h = wp.shape[1]

    def st_body(x_ref, asd_ref, st_ref):
        st_ref[...] = lax.dot_general(
            asd_ref[...], x_ref[...], (((1,), (1,)), ((), ())),
            precision=_HI, preferred_element_type=F32)

    st = pl.pallas_call(
        st_body,
        out_shape=jax.ShapeDtypeStruct((2, n), F32),
    )(x, asd)

    def body(x_ref, asd_ref, wp_ref, wq_ref, wu_ref, p_ref, q_ref, u_ref):
        xb = x_ref[...]
        p_ref[...] = _dot(xb, wp_ref[...])
        qb = _dot(xb, wq_ref[...])
        q_ref[...] = qb
        u_ref[...] = _dot(qb, wu_ref[...])

    p, q, u = pl.pallas_call(
        body,
        grid=(n // _NB,),
        in_specs=[
            pl.BlockSpec((_NB, d), lambda i: (i, 0)),
            pl.BlockSpec((2, d), lambda i: (0, 0)),
            pl.BlockSpec((d, h), lambda i: (0, 0)),
            pl.BlockSpec((d, h), lambda i: (0, 0)),
            pl.BlockSpec((h, h), lambda i: (0, 0)),
        ],
        out_specs=[
            pl.BlockSpec((_NB, h), lambda i: (i, 0)),
            pl.BlockSpec((_NB, h), lambda i: (i, 0)),
            pl.BlockSpec((_NB, h), lambda i: (i, 0)),
        ],
        out_shape=[
            jax.ShapeDtypeStruct((n, h), F32),
            jax.ShapeDtypeStruct((n, h), F32),
            jax.ShapeDtypeStruct((n, h), F32),
        ],
    )(x, asd, wp, wq, wu)
    return st, p, q, u


def _tc_mid_a(p, y, wd):
    n, h = p.shape

    def body(p_ref, y_ref, wd_ref, pn_ref, v_ref):
        pn = p_ref[...] + y_ref[0] + y_ref[1]
        pn_ref[...] = pn
        v_ref[...] = _dot(pn, wd_ref[...])

    return pl.pallas_call(
        body,
        grid=(n // _NB,),
        in_specs=[
            pl.BlockSpec((_NB, h), lambda i: (i, 0)),
            pl.BlockSpec((2, _NB, h), lambda i: (0, i, 0)),
            pl.BlockSpec((h, h), lambda i: (0, 0)),
        ],
        out_specs=[
            pl.BlockSpec((_NB, h), lambda i: (i, 0)),
            pl.BlockSpec((_NB, h), lambda i: (i, 0)),
        ],
        out_shape=[
            jax.ShapeDtypeStruct((n, h), F32),
            jax.ShapeDtypeStruct((n, h), F32),
        ],
    )(p, y, wd)


def _tc_mid_b(q, y, p, wu):
    n, h = q.shape

    def body(q_ref, y_ref, p_ref, wu_ref, pn_ref, qn_ref, u_ref):
        qn = jnp.tanh(q_ref[...] - (y_ref[0] + y_ref[1]))
        pn_ref[...] = jnp.tanh(p_ref[...])
        qn_ref[...] = qn
        u_ref[...] = _dot(qn, wu_ref[...])

    return pl.pallas_call(
        body,
        grid=(n // _NB,),
        in_specs=[
            pl.BlockSpec((_NB, h), lambda i: (i, 0)),
            pl.BlockSpec((2, _NB, h), lambda i: (0, i, 0)),
            pl.BlockSpec((_NB, h), lambda i: (i, 0)),
            pl.BlockSpec((h, h), lambda i: (0, 0)),
        ],
        out_specs=[
            pl.BlockSpec((_NB, h), lambda i: (i, 0)),
            pl.BlockSpec((_NB, h), lambda i: (i, 0)),
            pl.BlockSpec((_NB, h), lambda i: (i, 0)),
        ],
        out_shape=[
            jax.ShapeDtypeStruct((n, h), F32),
            jax.ShapeDtypeStruct((n, h), F32),
            jax.ShapeDtypeStruct((n, h), F32),
        ],
    )(q, y, p, wu)


def _tc_fin(q, y, wo, bo):
    n, h = q.shape
    c = wo.shape[1]

    def body(q_ref, y_ref, wo_ref, bo_ref, o_ref):
        qn = q_ref[...] - (y_ref[0] + y_ref[1])
        o_ref[...] = _dot(qn, wo_ref[...]) + bo_ref[...]

    return pl.pallas_call(
        body,
        grid=(n // _NB,),
        in_specs=[
            pl.BlockSpec((_NB, h), lambda i: (i, 0)),
            pl.BlockSpec((2, _NB, h), lambda i: (0, i, 0)),
            pl.BlockSpec((h, c), lambda i: (0, 0)),
            pl.BlockSpec((1, c), lambda i: (0, 0)),
        ],
        out_specs=pl.BlockSpec((_NB, c), lambda i: (i, 0)),
        out_shape=jax.ShapeDtypeStruct((n, c), F32),
    )(q, y, wo, bo)


_CH = 80


def _sc_softmax(st, src, dst):
    n = st.shape[1]
    e = src.shape[0]
    ec = e // NS
    ew = e // NW
    ch = _CH
    rc = ec // ch
    rw = ew // ch
    gpc = ch // L

    @functools.partial(
        pl.kernel,
        out_type=jax.ShapeDtypeStruct((e,), F32),
        mesh=_mesh(),
        compiler_params=pltpu.CompilerParams(needs_layout_passes=False),
        scratch_types=[
            pltpu.VMEM((n,), F32),
            pltpu.VMEM((n,), F32),
            pltpu.VMEM((ec,), I32),
            pltpu.VMEM((ec,), I32),
            pltpu.VMEM((ec,), F32),
            pltpu.VMEM((n,), F32),
            pltpu.VMEM((L,), F32),
            pltpu.VMEM((NS, L), F32),
            pltpu.SemaphoreType.DMA,
            pltpu.VMEM_SHARED((n,), F32),
            pltpu.VMEM_SHARED((NS, L), F32),
        ],
    )
    def k(st_hbm, src_hbm, dst_hbm, alpha_hbm, s_buf, t_buf, srcb, dstb, eb,
          dnm, mbuf, mall, ssem, dnm_sh, smax_sh):
        cid = lax.axis_index("c")
        sid = lax.axis_index("s")
        eoff = sid * ec
        pltpu.sync_copy(st_hbm.at[0], s_buf)
        pltpu.sync_copy(st_hbm.at[1], t_buf)
        pltpu.sync_copy(src_hbm.at[pl.ds(eoff, ec)], srcb)
        pltpu.sync_copy(dst_hbm.at[pl.ds(eoff, ec)], dstb)

        def zbody(i, _):
            for jz in range(8):
                dnm[pl.ds((i * 8 + jz) * L, L)] = jnp.zeros((L,), F32)
            return 0

        lax.fori_loop(0, n // (8 * L), zbody, 0)
        def ztail(i, _):
            dnm[pl.ds((n // (8 * L)) * 8 * L + i * L, L)] = jnp.zeros((L,), F32)
            return 0

        lax.fori_loop(0, (n // L) % 8, ztail, 0)
        zch = (n // NS) & ~7
        pltpu.sync_copy(dnm.at[pl.ds(sid * zch, zch)],
                        dnm_sh.at[pl.ds(sid * zch, zch)])
        rem = n - NS * zch
        if rem:
            @pl.when(sid == 0)
            def _():
                pltpu.sync_copy(dnm.at[pl.ds(NS * zch, rem)],
                                dnm_sh.at[pl.ds(NS * zch, rem)])

        def ebody(r, m):
            for g in range(gpc):
                o = r * ch + g * L
                sv = plsc.load_gather(s_buf, [srcb[pl.ds(o, L)]])
                tv = plsc.load_gather(t_buf, [dstb[pl.ds(o, L)]])
                ev = sv + tv
                ev = jnp.where(ev >= 0.0, ev, 0.2 * ev)
                eb[pl.ds(o, L)] = ev
                m = jnp.maximum(m, ev)
            return m

        m = lax.fori_loop(0, rc, ebody, jnp.full((L,), -jnp.inf, F32))
        mbuf[...] = m
        pltpu.sync_copy(mbuf, smax_sh.at[sid])
        plsc.subcore_barrier()
        pltpu.sync_copy(smax_sh, mall)
        mm = mall[0]
        for i in range(1, NS):
            mm = jnp.maximum(mm, mall[i])
        gmax = jnp.max(mm)

        def expbody(r, _):
            for g in range(gpc):
                ev = eb[pl.ds(r * ch + g * L, L)]
                eb[pl.ds(r * ch + g * L, L)] = jnp.exp(ev - gmax)
            return 0

        lax.fori_loop(0, rc, expbody, 0)
        bat = 25
        for b0 in range(rc // bat):
            for j in range(bat):
                r = b0 * bat + j
                pltpu.async_copy(eb.at[pl.ds(r * ch, ch)],
                                 dnm_sh.at[dstb.at[pl.ds(r * ch, ch)]],
                                 ssem, add=True)
            for j in range(bat):
                r = b0 * bat + j
                pltpu.make_async_copy(eb.at[pl.ds(r * ch, ch)],
                                      dnm_sh.at[dstb.at[pl.ds(r * ch, ch)]],
                                      ssem).wait()
        plsc.subcore_barrier()
        pltpu.sync_copy(dnm_sh, dnm)
        roff = cid * rw

        def abody(r, _):
            for g in range(gpc):
                o = (roff + r) * ch + g * L
                ee = eb[pl.ds(o, L)]
                dv = plsc.load_gather(dnm, [dstb[pl.ds(o, L)]])
                eb[pl.ds(o, L)] = ee / (dv + 1e-16)
            return 0

        lax.fori_loop(0, rw, abody, 0)
        pltpu.sync_copy(eb.at[pl.ds(roff * ch, ew)],
                        alpha_hbm.at[pl.ds(eoff + cid * ew, ew)])

    return k(st, src, dst)


def _sc_msg(u, src, dst, alpha):
    n, h = u.shape
    e = src.shape[0]
    ew = e // NW
    npad = -(-n // (NS * 128)) * (NS * 128)
    rpt = npad // NS
    zb = _CH
    ch = _CH
    nch = ew // ch
    hv = h // L

    sch = 25
    nsup = nch // sch

    @functools.partial(
        pl.kernel,
        out_type=jax.ShapeDtypeStruct((NC, npad, h), F32),
        mesh=_mesh(),
        compiler_params=pltpu.CompilerParams(needs_layout_passes=False),
        scratch_types=[
            pltpu.VMEM((sch * ch,), I32),
            pltpu.VMEM((sch * ch,), I32),
            pltpu.VMEM((sch * ch,), F32),
            pltpu.VMEM((ch, h), F32),
            pltpu.VMEM((ch, h), F32),
            pltpu.VMEM((ch, h), F32),
            pltpu.SemaphoreType.DMA,
            pltpu.SemaphoreType.DMA,
            pltpu.SemaphoreType.DMA,
            pltpu.SemaphoreType.DMA,
            pltpu.SemaphoreType.DMA,
            pltpu.SemaphoreType.DMA,
            pltpu.VMEM_SHARED((npad, h), F32),
        ],
    )
    def k(u_hbm, src_hbm, dst_hbm, a_hbm, out_hbm, srcb, dstb, ab,
          rowsA, rowsB, rowsC, gsA, gsB, gsC, ssA, ssB, ssC, y_sh):
        cid = lax.axis_index("c")
        sid = lax.axis_index("s")
        eoff = (cid * NS + sid) * ew

        def zbody(i, _):
            for jz in range(hv):
                rowsA[i, pl.ds(jz * L, L)] = jnp.zeros((L,), F32)
            return 0

        lax.fori_loop(0, zb, zbody, 0)
        for k2 in range(rpt // zb):
            pltpu.sync_copy(rowsA, y_sh.at[pl.ds(sid * rpt + k2 * zb, zb)])
        plsc.subcore_barrier()

        for sup in range(nsup):
            o0 = eoff + sup * sch * ch

            def fire(j, rows, gs):
                pltpu.async_copy(u_hbm.at[srcb.at[pl.ds(j * ch, ch)]], rows,
                                 gs)

            def swait(j, rows, ss):
                pltpu.make_async_copy(
                    rows, y_sh.at[dstb.at[pl.ds(j * ch, ch)]], ss).wait()

            def proc(j, rows, gs, ss, nrows, ngs, nss):
                pltpu.make_async_copy(
                    u_hbm.at[srcb.at[pl.ds(j * ch, ch)]], rows, gs).wait()

                def sbody(g, _2):
                    av16 = ab[pl.ds(j * ch + g * L, L)]
                    for kk in range(L):
                        av = av16[kk]
                        for jj in range(hv):
                            sl = pl.ds(jj * L, L)
                            rows[g * L + kk, sl] = rows[g * L + kk, sl] * av
                    return 0

                lax.fori_loop(0, ch // L, sbody, 0)
                pltpu.async_copy(rows, y_sh.at[dstb.at[pl.ds(j * ch, ch)]],
                                 ss, add=True)

                @pl.when(j + 2 < sch)
                def _():
                    @pl.when(j >= 1)
                    def _():
                        swait(j - 1, nrows, nss)

                    fire(j + 2, nrows, ngs)

            pltpu.sync_copy(src_hbm.at[pl.ds(o0, sch * ch)], srcb)
            pltpu.sync_copy(dst_hbm.at[pl.ds(o0, sch * ch)], dstb)
            pltpu.sync_copy(a_hbm.at[pl.ds(o0, sch * ch)], ab)
            fire(0, rowsA, gsA)
            fire(1, rowsB, gsB)
            bufs = [(rowsA, gsA, ssA), (rowsB, gsB, ssB), (rowsC, gsC, ssC)]

            def pbody(j, _):
                for par in range(3):
                    @pl.when(j % 3 == par)
                    def _(par=par):
                        rows, gs, ss = bufs[par]
                        nrows, ngs, nss = bufs[(par + 2) % 3]
                        proc(j, rows, gs, ss, nrows, ngs, nss)

                return 0

            lax.fori_loop(0, sch, pbody, 0)
            for jj in range(sch - 3, sch):
                rows, gs, ss = bufs[jj % 3]
                swait(jj, rows, ss)
        plsc.subcore_barrier()
        pltpu.sync_copy(y_sh.at[pl.ds(sid * rpt, rpt)],
                        out_hbm.at[cid].at[pl.ds(sid * rpt, rpt)])

    return k(u, src, dst, alpha)


def kernel(x, edge_index, a_src, a_dst, W_p, W_q, W_up0, W_up1, W_up2,
           W_down0, W_down1, W_down2, W_out, b_out):
    src = edge_index[0]
    dst = edge_index[1]
    asd = jnp.stack([a_src, a_dst], axis=0)
    st, p, q, u = _tc_pre(x, asd, W_p, W_q, W_up0)
    alpha = _sc_softmax(st, src, dst)
    ups = [W_up0, W_up1, W_up2]
    downs = [W_down0, W_down1, W_down2]
    out = None
    for i in range(3):
        yp = _sc_msg(u, src, dst, alpha)
        p, v = _tc_mid_a(p, yp, downs[i])
        yq = _sc_msg(v, src, dst, alpha)
        if i < 2:
            p, q, u = _tc_mid_b(q, yq, p, ups[i + 1])
        else:
            out = _tc_fin(q, yq, W_out, b_out.reshape(1, -1))
    return out
